# baseline (speedup 1.0000x reference)
import functools

import jax
import jax.numpy as jnp
from jax import lax
from jax.experimental import pallas as pl
from jax.experimental.pallas import tpu as pltpu
from jax.experimental.pallas import tpu_sc as plsc

BATCH = 16384
HIST = 50
D = 64
VOCAB = 100001
NW = 32
PASSES = D // NW
CB = 512
NKB = BATCH // CB
HB = 8
NHB = HIST // HB
HTAIL = HIST - NHB * HB

_mesh = plsc.VectorSubcoreMesh(core_axis_name="c", subcore_axis_name="s")


@functools.partial(
    pl.kernel,
    mesh=_mesh,
    out_type=jax.ShapeDtypeStruct((HIST, D, BATCH), jnp.float32),
    compiler_params=pltpu.CompilerParams(needs_layout_passes=False),
    scratch_types=[
        pltpu.VMEM((VOCAB,), jnp.float32),
        pltpu.VMEM((HB, CB), jnp.int32),
        pltpu.VMEM((HB, CB), jnp.int32),
        pltpu.VMEM((HB, CB), jnp.float32),
        pltpu.VMEM((HB, CB), jnp.float32),
        pltpu.SemaphoreType.DMA,
        pltpu.SemaphoreType.DMA,
        pltpu.SemaphoreType.DMA,
        pltpu.SemaphoreType.DMA,
    ],
)
def _sc_gather_t(table_hbm, idx_hbm, out_hbm, trow,
                 ib0, ib1, ob0, ob1, i0, i1, s0, s1):
    ibuf = (ib0, ib1)
    obuf = (ob0, ob1)
    isem = (i0, i1)
    ssem = (s0, s1)
    wid = lax.axis_index("s") * 2 + lax.axis_index("c")

    def one_pass(p, carry):
        f = wid * PASSES + p
        pltpu.sync_copy(table_hbm.at[f], trow)

        def run_hblock(h0, nr):
            def idxcp(kb, s):
                return pltpu.make_async_copy(
                    idx_hbm.at[pl.ds(h0, nr), pl.ds(kb * CB, CB)],
                    ibuf[s].at[pl.ds(0, nr)], isem[s])

            def stcp(kb, s):
                return pltpu.make_async_copy(
                    obuf[s].at[pl.ds(0, nr)],
                    out_hbm.at[pl.ds(h0, nr), f, pl.ds(kb * CB, CB)],
                    ssem[s])

            def gather(s):
                def row(r, cr):
                    @plsc.parallel_loop(0, CB, step=16, unroll=16)
                    def body(c):
                        iv = ibuf[s][r, pl.ds(c, 16)]
                        obuf[s][r, pl.ds(c, 16)] = plsc.load_gather(trow, [iv])
                    return cr
                lax.fori_loop(0, nr, row, 0)

            def visit(kb, s, pref, swait):
                if swait:
                    stcp(kb - 2, s).wait()
                idxcp(kb, s).wait()
                gather(s)
                stcp(kb, s).start()
                if pref:
                    idxcp(kb + 2, s).start()

            idxcp(0, 0).start()
            idxcp(1, 1).start()
            visit(0, 0, True, False)
            visit(1, 1, True, False)

            def pair(g, cr):
                visit(g, 0, True, True)
                visit(g + 1, 1, True, True)
                return cr

            lax.fori_loop(1, NKB // 2 - 1, lambda i, cr: pair(i * 2, cr), 0)
            visit(NKB - 2, 0, False, True)
            visit(NKB - 1, 1, False, True)
            stcp(NKB - 2, 0).wait()
            stcp(NKB - 1, 1).wait()

        def hblock(hb, cr):
            run_hblock(hb * HB, HB)
            return cr

        lax.fori_loop(0, NHB, hblock, 0)
        run_hblock(NHB * HB, HTAIL)
        return carry

    lax.fori_loop(0, PASSES, one_pass, 0)


def kernel(input, word_embed_weight):
    table_t = word_embed_weight.T
    idx_t = input.T
    out_t = _sc_gather_t(table_t, idx_t)
    return jnp.transpose(out_t, (2, 0, 1))


# speedup vs baseline: 1.0385x; 1.0385x over previous
import functools

import jax
import jax.numpy as jnp
from jax import lax
from jax.experimental import pallas as pl
from jax.experimental.pallas import tpu as pltpu
from jax.experimental.pallas import tpu_sc as plsc

BATCH = 16384
HIST = 50
D = 64
VOCAB = 100001
NW = 32
PASSES = D // NW
CB = 512
NKB = BATCH // CB
HB = 8
NHB = HIST // HB
HTAIL = HIST - NHB * HB

_mesh = plsc.VectorSubcoreMesh(core_axis_name="c", subcore_axis_name="s")


@functools.partial(
    pl.kernel,
    mesh=_mesh,
    out_type=jax.ShapeDtypeStruct((HIST, D, BATCH), jnp.float32),
    compiler_params=pltpu.CompilerParams(needs_layout_passes=False),
    scratch_types=[
        pltpu.VMEM((VOCAB,), jnp.float32),
        pltpu.VMEM((HB, CB), jnp.int32),
        pltpu.VMEM((HB, CB), jnp.int32),
        pltpu.VMEM((HB, CB), jnp.float32),
        pltpu.VMEM((HB, CB), jnp.float32),
        pltpu.SemaphoreType.DMA,
        pltpu.SemaphoreType.DMA,
        pltpu.SemaphoreType.DMA,
        pltpu.SemaphoreType.DMA,
    ],
)
def _sc_gather_t(table_hbm, idx_hbm, out_hbm, trow,
                 ib0, ib1, ob0, ob1, i0, i1, s0, s1):
    ibuf = (ib0, ib1)
    obuf = (ob0, ob1)
    isem = (i0, i1)
    ssem = (s0, s1)
    wid = lax.axis_index("s") * 2 + lax.axis_index("c")

    def one_pass(p, carry):
        f = wid * PASSES + p
        pltpu.sync_copy(table_hbm.at[f], trow)

        def run_hblock(h0, nr):
            def idxcp(kb, s):
                return pltpu.make_async_copy(
                    idx_hbm.at[pl.ds(h0, nr), pl.ds(kb * CB, CB)],
                    ibuf[s].at[pl.ds(0, nr)], isem[s])

            def stcp(kb, s):
                return pltpu.make_async_copy(
                    obuf[s].at[pl.ds(0, nr)],
                    out_hbm.at[pl.ds(h0, nr), f, pl.ds(kb * CB, CB)],
                    ssem[s])

            def gather(s):
                def row(r, cr):
                    @plsc.parallel_loop(0, CB, step=16, unroll=8)
                    def body(c):
                        iv = ibuf[s][r, pl.ds(c, 16)]
                        obuf[s][r, pl.ds(c, 16)] = plsc.load_gather(trow, [iv])
                    return cr
                lax.fori_loop(0, nr, row, 0)

            def visit(kb, s, pref, swait):
                if swait:
                    stcp(kb - 2, s).wait()
                idxcp(kb, s).wait()
                gather(s)
                stcp(kb, s).start()
                if pref:
                    idxcp(kb + 2, s).start()

            idxcp(0, 0).start()
            idxcp(1, 1).start()
            visit(0, 0, True, False)
            visit(1, 1, True, False)

            def pair(g, cr):
                visit(g, 0, True, True)
                visit(g + 1, 1, True, True)
                return cr

            lax.fori_loop(1, NKB // 2 - 1, lambda i, cr: pair(i * 2, cr), 0)
            visit(NKB - 2, 0, False, True)
            visit(NKB - 1, 1, False, True)
            stcp(NKB - 2, 0).wait()
            stcp(NKB - 1, 1).wait()

        def hblock(hb, cr):
            run_hblock(hb * HB, HB)
            return cr

        lax.fori_loop(0, NHB, hblock, 0)
        run_hblock(NHB * HB, HTAIL)
        return carry

    lax.fori_loop(0, PASSES, one_pass, 0)


def kernel(input, word_embed_weight):
    table_t = word_embed_weight.T
    idx_t = input.T
    out_t = _sc_gather_t(table_t, idx_t)
    return jnp.transpose(out_t, (2, 0, 1))
